# baseline (speedup 1.0000x reference)
import functools

import jax
import jax.numpy as jnp
from jax.experimental import pallas as pl
from jax.experimental.pallas import tpu as pltpu

B = 1
L = 2048
H = 768
F = 4 * H
E = 8
TOPK = 2
AUX_COEF = 0.01

BT = 256


def _router_body(x_ref, gw_ref, combine_ref, aux_ref):
    x = x_ref[...]
    gw = gw_ref[...]
    logits = jax.lax.dot_general(
        x, gw, (((1,), (1,)), ((), ())),
        preferred_element_type=jnp.float32)
    m = jnp.max(logits, axis=-1, keepdims=True)
    ex = jnp.exp(logits - m)
    probs = ex / jnp.sum(ex, axis=-1, keepdims=True)

    lane = jax.lax.broadcasted_iota(jnp.int32, (L, E), 1)
    p0 = jnp.max(probs, axis=-1, keepdims=True)
    is0 = (probs == p0)
    e0 = jnp.min(jnp.where(is0, lane, E), axis=-1, keepdims=True)
    probs1 = jnp.where(lane == e0, -jnp.inf, probs)
    p1 = jnp.max(probs1, axis=-1, keepdims=True)
    is1 = (probs1 == p1)
    e1 = jnp.min(jnp.where(is1, lane, E), axis=-1, keepdims=True)

    s = p0 + p1
    w0 = p0 / s
    w1 = p1 / s
    oh0 = (lane == e0).astype(jnp.float32)
    oh1 = (lane == e1).astype(jnp.float32)
    combine = w0 * oh0 + w1 * oh1
    combine_ref[...] = combine

    counts = jnp.sum(oh0 + oh1, axis=0)
    f = counts / (L * TOPK)
    pmean = jnp.mean(probs, axis=0)
    aux = E * jnp.sum(f * pmean) * AUX_COEF
    aux_ref[...] = jnp.reshape(aux, (1, 1))


def _ffn_body(x_ref, fc_ref, pj_ref, cmb_ref, out_ref):
    e = pl.program_id(1)

    @pl.when(e == 0)
    def _init():
        out_ref[...] = jnp.zeros_like(out_ref)

    x = x_ref[...]
    h1 = jax.lax.dot_general(
        x, fc_ref[0], (((1,), (1,)), ((), ())),
        preferred_element_type=jnp.float32)
    h1 = jnp.square(jnp.maximum(h1, 0.0))
    h2 = jax.lax.dot_general(
        h1, pj_ref[0], (((1,), (1,)), ((), ())),
        preferred_element_type=jnp.float32)
    lane = jax.lax.broadcasted_iota(jnp.int32, (BT, E), 1)
    w = jnp.sum(jnp.where(lane == e, cmb_ref[...], 0.0), axis=1, keepdims=True)
    out_ref[...] += h2 * w


def kernel(x, gate_W, fc_W, proj_W):
    x2 = x.reshape(L, H)
    combine, aux = pl.pallas_call(
        _router_body,
        out_shape=(
            jax.ShapeDtypeStruct((L, E), jnp.float32),
            jax.ShapeDtypeStruct((1, 1), jnp.float32),
        ),
        in_specs=[
            pl.BlockSpec(memory_space=pltpu.VMEM),
            pl.BlockSpec(memory_space=pltpu.VMEM),
        ],
        out_specs=(
            pl.BlockSpec(memory_space=pltpu.VMEM),
            pl.BlockSpec(memory_space=pltpu.VMEM),
        ),
    )(x2, gate_W)

    nb = L // BT
    out = pl.pallas_call(
        _ffn_body,
        grid=(nb, E),
        in_specs=[
            pl.BlockSpec((BT, H), lambda t, e: (t, 0)),
            pl.BlockSpec((1, F, H), lambda t, e: (e, 0, 0)),
            pl.BlockSpec((1, H, F), lambda t, e: (e, 0, 0)),
            pl.BlockSpec((BT, E), lambda t, e: (t, 0)),
        ],
        out_specs=pl.BlockSpec((BT, H), lambda t, e: (t, 0)),
        out_shape=jax.ShapeDtypeStruct((L, H), jnp.float32),
        compiler_params=pltpu.CompilerParams(
            dimension_semantics=("arbitrary", "arbitrary"),
        ),
    )(x2, fc_W, proj_W, combine)

    return out.reshape(B, L, H), aux.reshape(())


# speedup vs baseline: 2.4223x; 2.4223x over previous
import functools

import jax
import jax.numpy as jnp
from jax import lax
from jax.experimental import pallas as pl
from jax.experimental.pallas import tpu as pltpu
from jax.experimental.pallas import tpu_sc as plsc

B = 1
L = 2048
H = 768
F = 4 * H
E = 8
TOPK = 2
AUX_COEF = 0.01

BT = 256
NB = 23
PPAD = NB * BT
NW = 32
TPB = L // NW
CH = 256


def _router_body(x_ref, gw_ref, p0_ref, p1_ref, w0_ref, w1_ref,
                 be_ref, bv_ref, aux_ref):
    x = x_ref[...]
    gw = gw_ref[...]
    logits = lax.dot_general(x, gw, (((1,), (1,)), ((), ())),
                             preferred_element_type=jnp.float32)
    m = jnp.max(logits, axis=-1, keepdims=True)
    ex = jnp.exp(logits - m)
    probs = ex / jnp.sum(ex, axis=-1, keepdims=True)

    lane = lax.broadcasted_iota(jnp.int32, (L, E), 1)
    v0 = jnp.max(probs, axis=-1, keepdims=True)
    e0 = jnp.min(jnp.where(probs == v0, lane, E), axis=-1, keepdims=True)
    probs1 = jnp.where(lane == e0, -jnp.inf, probs)
    v1 = jnp.max(probs1, axis=-1, keepdims=True)
    e1 = jnp.min(jnp.where(probs1 == v1, lane, E), axis=-1, keepdims=True)

    s = v0 + v1
    w0 = v0 / s
    w1 = v1 / s
    oh0 = (lane == e0).astype(jnp.float32)
    oh1 = (lane == e1).astype(jnp.float32)

    r = lax.broadcasted_iota(jnp.int32, (CH, CH), 0)
    c = lax.broadcasted_iota(jnp.int32, (CH, CH), 1)
    tri = (c < r).astype(jnp.float32)
    exc0_chunks = []
    exc1_chunks = []
    carry0 = jnp.zeros((1, E), jnp.float32)
    carry1 = jnp.zeros((1, E), jnp.float32)
    for i in range(L // CH):
        o0 = oh0[i * CH:(i + 1) * CH]
        o1 = oh1[i * CH:(i + 1) * CH]
        exc0_chunks.append(
            lax.dot_general(tri, o0, (((1,), (0,)), ((), ())),
                            preferred_element_type=jnp.float32) + carry0)
        exc1_chunks.append(
            lax.dot_general(tri, o1, (((1,), (0,)), ((), ())),
                            preferred_element_type=jnp.float32) + carry1)
        carry0 = carry0 + jnp.sum(o0, axis=0, keepdims=True)
        carry1 = carry1 + jnp.sum(o1, axis=0, keepdims=True)
    exc0 = jnp.concatenate(exc0_chunks, axis=0)
    exc1 = jnp.concatenate(exc1_chunks, axis=0)
    counts0 = carry0
    counts1 = carry1
    counts = counts0 + counts1

    pc = jnp.ceil(counts / BT) * BT
    er = lax.broadcasted_iota(jnp.int32, (E, E), 0)
    ec = lax.broadcasted_iota(jnp.int32, (E, E), 1)
    ut = (er <= ec).astype(jnp.float32)
    ends = lax.dot_general(pc, ut, (((1,), (0,)), ((), ())),
                           preferred_element_type=jnp.float32)
    starts = ends - pc

    startsb = jnp.broadcast_to(starts, (L, E))
    c0b = jnp.broadcast_to(counts0, (L, E))
    sel0 = (lane == e0)
    sel1 = (lane == e1)
    pos0 = jnp.sum(jnp.where(sel0, startsb + exc0, 0.0), axis=-1,
                   keepdims=True)
    pos1 = jnp.sum(jnp.where(sel1, startsb + c0b + exc1, 0.0), axis=-1,
                   keepdims=True)
    p0_ref[...] = pos0.astype(jnp.int32)
    p1_ref[...] = pos1.astype(jnp.int32)
    w0_ref[...] = jnp.broadcast_to(w0, (L, 16))
    w1_ref[...] = jnp.broadcast_to(w1, (L, 16))

    bids = (lax.broadcasted_iota(jnp.int32, (1, NB), 1) * BT).astype(
        jnp.float32)
    total = ends[0, E - 1]
    be = jnp.zeros((1, NB), jnp.int32)
    last_e = jnp.int32(0)
    for e in range(E):
        be = be + (bids >= ends[0, e]).astype(jnp.int32)
        last_e = last_e + (ends[0, e] <= total - 1.0).astype(jnp.int32)
    valid = (bids < total)
    be_ref[...] = jnp.where(valid, jnp.minimum(be, E - 1), last_e)
    bv_ref[...] = valid.astype(jnp.int32)

    f = counts / (L * TOPK)
    pmean = jnp.mean(probs, axis=0, keepdims=True)
    aux = E * jnp.sum(f * pmean) * AUX_COEF
    aux_ref[...] = jnp.reshape(aux, (1, 1))


@functools.lru_cache(maxsize=None)
def _sc_kernels():
    mesh = plsc.VectorSubcoreMesh(core_axis_name="c", subcore_axis_name="s")

    @functools.partial(
        pl.kernel,
        out_type=jax.ShapeDtypeStruct((PPAD, H), jnp.float32),
        mesh=mesh,
        scratch_types=[
            pltpu.VMEM((TPB,), jnp.int32),
            pltpu.VMEM((TPB,), jnp.int32),
            pltpu.VMEM((TPB, H), jnp.float32),
            pltpu.SemaphoreType.DMA,
            pltpu.SemaphoreType.DMA,
        ],
    )
    def _dispatch(x_hbm, p0_hbm, p1_hbm, xs_hbm, idx0_v, idx1_v, xv,
                  sem0, sem1):
        wid = lax.axis_index("s") * 2 + lax.axis_index("c")
        base = wid * TPB
        pltpu.sync_copy(p0_hbm.at[pl.ds(base, TPB)], idx0_v)
        pltpu.sync_copy(p1_hbm.at[pl.ds(base, TPB)], idx1_v)
        pltpu.sync_copy(x_hbm.at[pl.ds(base, TPB)], xv)
        c0 = pltpu.async_copy(xv, xs_hbm.at[idx0_v], sem0)
        c1 = pltpu.async_copy(xv, xs_hbm.at[idx1_v], sem1)
        c0.wait()
        c1.wait()

    @functools.partial(
        pl.kernel,
        out_type=jax.ShapeDtypeStruct((L, H), jnp.float32),
        mesh=mesh,
        scratch_types=[
            pltpu.VMEM((TPB,), jnp.int32),
            pltpu.VMEM((TPB,), jnp.int32),
            pltpu.VMEM((TPB, 16), jnp.float32),
            pltpu.VMEM((TPB, 16), jnp.float32),
            pltpu.VMEM((TPB, H), jnp.float32),
            pltpu.VMEM((TPB, H), jnp.float32),
            pltpu.SemaphoreType.DMA,
            pltpu.SemaphoreType.DMA,
        ],
    )
    def _combine(ys_hbm, p0_hbm, p1_hbm, w0_hbm, w1_hbm, out_hbm,
                 idx0_v, idx1_v, wv0, wv1, b0, b1, sem0, sem1):
        wid = lax.axis_index("s") * 2 + lax.axis_index("c")
        base = wid * TPB
        pltpu.sync_copy(p0_hbm.at[pl.ds(base, TPB)], idx0_v)
        pltpu.sync_copy(p1_hbm.at[pl.ds(base, TPB)], idx1_v)
        pltpu.sync_copy(w0_hbm.at[pl.ds(base, TPB)], wv0)
        pltpu.sync_copy(w1_hbm.at[pl.ds(base, TPB)], wv1)
        g0 = pltpu.async_copy(ys_hbm.at[idx0_v], b0, sem0)
        g1 = pltpu.async_copy(ys_hbm.at[idx1_v], b1, sem1)
        g0.wait()
        g1.wait()

        def row(j, _):
            a0 = wv0[j, :]
            a1 = wv1[j, :]
            for cc in range(H // 16):
                sl = pl.ds(cc * 16, 16)
                b0[j, sl] = a0 * b0[j, sl] + a1 * b1[j, sl]
            return 0

        lax.fori_loop(0, TPB, row, 0)
        pltpu.sync_copy(b0, out_hbm.at[pl.ds(base, TPB)])

    return _dispatch, _combine


def _ffn_body(be_ref, bv_ref, xs_ref, fc_ref, pj_ref, out_ref):
    b = pl.program_id(0)

    @pl.when(bv_ref[0, b] == 1)
    def _():
        x = xs_ref[...]
        h1 = lax.dot_general(x, fc_ref[0], (((1,), (1,)), ((), ())),
                             preferred_element_type=jnp.float32)
        h1 = jnp.square(jnp.maximum(h1, 0.0))
        out_ref[...] = lax.dot_general(
            h1, pj_ref[0], (((1,), (1,)), ((), ())),
            preferred_element_type=jnp.float32)


def kernel(x, gate_W, fc_W, proj_W):
    x2 = x.reshape(L, H)
    p0, p1, w0, w1, be, bv, aux = pl.pallas_call(
        _router_body,
        out_shape=(
            jax.ShapeDtypeStruct((L, 1), jnp.int32),
            jax.ShapeDtypeStruct((L, 1), jnp.int32),
            jax.ShapeDtypeStruct((L, 16), jnp.float32),
            jax.ShapeDtypeStruct((L, 16), jnp.float32),
            jax.ShapeDtypeStruct((1, NB), jnp.int32),
            jax.ShapeDtypeStruct((1, NB), jnp.int32),
            jax.ShapeDtypeStruct((1, 1), jnp.float32),
        ),
    )(x2, gate_W)
    p0f = p0.reshape(L)
    p1f = p1.reshape(L)

    _dispatch, _combine = _sc_kernels()
    xs = _dispatch(x2, p0f, p1f)

    ys = pl.pallas_call(
        _ffn_body,
        grid_spec=pltpu.PrefetchScalarGridSpec(
            num_scalar_prefetch=2,
            grid=(NB,),
            in_specs=[
                pl.BlockSpec((BT, H), lambda b, be, bv: (b, 0)),
                pl.BlockSpec((1, F, H), lambda b, be, bv: (be[0, b], 0, 0)),
                pl.BlockSpec((1, H, F), lambda b, be, bv: (be[0, b], 0, 0)),
            ],
            out_specs=pl.BlockSpec((BT, H), lambda b, be, bv: (b, 0)),
        ),
        out_shape=jax.ShapeDtypeStruct((PPAD, H), jnp.float32),
        compiler_params=pltpu.CompilerParams(
            dimension_semantics=("arbitrary",),
        ),
    )(be, bv, xs, fc_W, proj_W)

    out = _combine(ys, p0f, p1f, w0, w1)
    return out.reshape(B, L, H), aux.reshape(())


# speedup vs baseline: 2.7995x; 1.1557x over previous
import functools

import jax
import jax.numpy as jnp
from jax import lax
from jax.experimental import pallas as pl
from jax.experimental.pallas import tpu as pltpu
from jax.experimental.pallas import tpu_sc as plsc

B = 1
L = 2048
H = 768
F = 4 * H
E = 8
TOPK = 2
AUX_COEF = 0.01

BT = 256
NB = 23
PPAD = NB * BT
NW = 32
TPB = L // NW
CH = 256


def _router_body(x_ref, gw_ref, p0_ref, p1_ref, w0_ref, w1_ref,
                 bv_ref, er_ref, ue_ref, nu_ref, aux_ref):
    x = x_ref[...]
    gw = gw_ref[...]
    logits = lax.dot_general(x, gw, (((1,), (1,)), ((), ())),
                             preferred_element_type=jnp.float32)
    m = jnp.max(logits, axis=-1, keepdims=True)
    ex = jnp.exp(logits - m)
    probs = ex / jnp.sum(ex, axis=-1, keepdims=True)

    lane = lax.broadcasted_iota(jnp.int32, (L, E), 1)
    v0 = jnp.max(probs, axis=-1, keepdims=True)
    e0 = jnp.min(jnp.where(probs == v0, lane, E), axis=-1, keepdims=True)
    probs1 = jnp.where(lane == e0, -jnp.inf, probs)
    v1 = jnp.max(probs1, axis=-1, keepdims=True)
    e1 = jnp.min(jnp.where(probs1 == v1, lane, E), axis=-1, keepdims=True)

    s = v0 + v1
    w0 = v0 / s
    w1 = v1 / s
    oh0 = (lane == e0).astype(jnp.float32)
    oh1 = (lane == e1).astype(jnp.float32)

    r = lax.broadcasted_iota(jnp.int32, (CH, CH), 0)
    c = lax.broadcasted_iota(jnp.int32, (CH, CH), 1)
    tri = (c < r).astype(jnp.float32)
    oh = jnp.concatenate([oh0, oh1], axis=1)
    exc_chunks = []
    carry = jnp.zeros((1, 2 * E), jnp.float32)
    for i in range(L // CH):
        o = oh[i * CH:(i + 1) * CH]
        exc_chunks.append(
            lax.dot_general(tri, o, (((1,), (0,)), ((), ())),
                            preferred_element_type=jnp.float32) + carry)
        carry = carry + jnp.sum(o, axis=0, keepdims=True)
    exc = jnp.concatenate(exc_chunks, axis=0)
    exc0 = exc[:, :E]
    exc1 = exc[:, E:]
    counts0 = carry[:, :E]
    counts1 = carry[:, E:]
    counts = counts0 + counts1

    pc = jnp.ceil(counts / BT) * BT
    er_ = lax.broadcasted_iota(jnp.int32, (E, E), 0)
    ec_ = lax.broadcasted_iota(jnp.int32, (E, E), 1)
    ut = (er_ <= ec_).astype(jnp.float32)
    ends = lax.dot_general(pc, ut, (((1,), (0,)), ((), ())),
                           preferred_element_type=jnp.float32)
    starts = ends - pc

    startsb = jnp.broadcast_to(starts, (L, E))
    c0b = jnp.broadcast_to(counts0, (L, E))
    sel0 = (lane == e0)
    sel1 = (lane == e1)
    pos0 = jnp.sum(jnp.where(sel0, startsb + exc0, 0.0), axis=-1,
                   keepdims=True)
    pos1 = jnp.sum(jnp.where(sel1, startsb + c0b + exc1, 0.0), axis=-1,
                   keepdims=True)
    p0_ref[...] = pos0.astype(jnp.int32)
    p1_ref[...] = pos1.astype(jnp.int32)
    w0_ref[...] = jnp.broadcast_to(w0, (L, 16))
    w1_ref[...] = jnp.broadcast_to(w1, (L, 16))

    bids = (lax.broadcasted_iota(jnp.int32, (1, NB), 1) * BT).astype(
        jnp.float32)
    total = ends[0, E - 1]
    valid = (bids < total)
    bv_ref[...] = valid.astype(jnp.int32)

    acc = jnp.zeros((1, NB), jnp.int32)
    for e in range(E):
        nz_e = pc[0:1, e:e + 1] > 0.0
        cond = jnp.logical_and(starts[0:1, e:e + 1] <= bids, nz_e)
        acc = acc + cond.astype(jnp.int32)
    er_ref[...] = jnp.maximum(acc - 1, 0)

    lane_r = lax.broadcasted_iota(jnp.int32, (1, E), 1)
    ue = jnp.zeros((1, E), jnp.int32)
    runn = jnp.zeros((1, 1), jnp.float32)
    for e in range(E):
        nz_e = pc[0:1, e:e + 1] > 0.0
        hit = jnp.logical_and(lane_r == runn.astype(jnp.int32), nz_e)
        ue = ue + jnp.where(hit, e, 0)
        runn = runn + nz_e.astype(jnp.float32)
    ue_ref[...] = ue
    nu_ref[...] = runn.astype(jnp.int32)

    f = counts / (L * TOPK)
    pmean = jnp.mean(probs, axis=0, keepdims=True)
    aux = E * jnp.sum(f * pmean) * AUX_COEF
    aux_ref[...] = jnp.reshape(aux, (1, 1))


@functools.lru_cache(maxsize=None)
def _sc_kernels():
    mesh = plsc.VectorSubcoreMesh(core_axis_name="c", subcore_axis_name="s")

    @functools.partial(
        pl.kernel,
        out_type=jax.ShapeDtypeStruct((PPAD, H), jnp.float32),
        mesh=mesh,
        scratch_types=[
            pltpu.VMEM((TPB,), jnp.int32),
            pltpu.VMEM((TPB,), jnp.int32),
            pltpu.VMEM((TPB, H), jnp.float32),
            pltpu.SemaphoreType.DMA,
            pltpu.SemaphoreType.DMA,
        ],
    )
    def _dispatch(x_hbm, p0_hbm, p1_hbm, xs_hbm, idx0_v, idx1_v, xv,
                  sem0, sem1):
        wid = lax.axis_index("s") * 2 + lax.axis_index("c")
        base = wid * TPB
        pltpu.sync_copy(p0_hbm.at[pl.ds(base, TPB)], idx0_v)
        pltpu.sync_copy(p1_hbm.at[pl.ds(base, TPB)], idx1_v)
        pltpu.sync_copy(x_hbm.at[pl.ds(base, TPB)], xv)
        c0 = pltpu.async_copy(xv, xs_hbm.at[idx0_v], sem0)
        c1 = pltpu.async_copy(xv, xs_hbm.at[idx1_v], sem1)
        c0.wait()
        c1.wait()

    @functools.partial(
        pl.kernel,
        out_type=jax.ShapeDtypeStruct((L, H), jnp.float32),
        mesh=mesh,
        scratch_types=[
            pltpu.VMEM((TPB,), jnp.int32),
            pltpu.VMEM((TPB,), jnp.int32),
            pltpu.VMEM((TPB, 16), jnp.float32),
            pltpu.VMEM((TPB, 16), jnp.float32),
            pltpu.VMEM((TPB, H), jnp.float32),
            pltpu.VMEM((TPB, H), jnp.float32),
            pltpu.SemaphoreType.DMA,
            pltpu.SemaphoreType.DMA,
        ],
    )
    def _combine(ys_hbm, p0_hbm, p1_hbm, w0_hbm, w1_hbm, out_hbm,
                 idx0_v, idx1_v, wv0, wv1, b0, b1, sem0, sem1):
        wid = lax.axis_index("s") * 2 + lax.axis_index("c")
        base = wid * TPB
        pltpu.sync_copy(p0_hbm.at[pl.ds(base, TPB)], idx0_v)
        pltpu.sync_copy(p1_hbm.at[pl.ds(base, TPB)], idx1_v)
        pltpu.sync_copy(w0_hbm.at[pl.ds(base, TPB)], wv0)
        pltpu.sync_copy(w1_hbm.at[pl.ds(base, TPB)], wv1)
        g0 = pltpu.async_copy(ys_hbm.at[idx0_v], b0, sem0)
        g1 = pltpu.async_copy(ys_hbm.at[idx1_v], b1, sem1)
        g0.wait()
        g1.wait()

        def row(j, _):
            a0 = wv0[j, :]
            a1 = wv1[j, :]
            for cc in range(H // 16):
                sl = pl.ds(cc * 16, 16)
                b0[j, sl] = a0 * b0[j, sl] + a1 * b1[j, sl]
            return 0

        lax.fori_loop(0, TPB, row, 0)
        pltpu.sync_copy(b0, out_hbm.at[pl.ds(base, TPB)])

    return _dispatch, _combine


def _ffn_body(bv_ref, er_ref, ue_ref, nu_ref, xs_ref, fc_ref, pj_ref,
              out_ref, fcv, pjv, sems):
    b = pl.program_id(0)
    r = er_ref[0, b]
    slot = lax.rem(r, 2)
    nu = nu_ref[0, 0]
    prev_r = er_ref[0, jnp.maximum(b - 1, 0)]
    first_use = jnp.logical_or(b == 0, r > prev_r)

    def start_copy(rank):
        eid = ue_ref[0, rank]
        sl = lax.rem(rank, 2)
        pltpu.make_async_copy(fc_ref.at[eid], fcv.at[sl], sems.at[sl]).start()
        pltpu.make_async_copy(pj_ref.at[eid], pjv.at[sl], sems.at[sl]).start()

    @pl.when(b == 0)
    def _():
        start_copy(r)

        @pl.when(nu > 1)
        def _():
            start_copy(r + 1)

    @pl.when(jnp.logical_and(jnp.logical_and(b > 0, r > prev_r), r + 1 < nu))
    def _():
        start_copy(r + 1)

    @pl.when(first_use)
    def _():
        pltpu.make_async_copy(fc_ref.at[0], fcv.at[slot],
                              sems.at[slot]).wait()
        pltpu.make_async_copy(pj_ref.at[0], pjv.at[slot],
                              sems.at[slot]).wait()

    @pl.when(bv_ref[0, b] == 1)
    def _():
        x = xs_ref[...]
        h1 = lax.dot_general(x, fcv[slot], (((1,), (1,)), ((), ())),
                             preferred_element_type=jnp.float32)
        h1 = jnp.square(jnp.maximum(h1, 0.0))
        out_ref[...] = lax.dot_general(
            h1, pjv[slot], (((1,), (1,)), ((), ())),
            preferred_element_type=jnp.float32)


def kernel(x, gate_W, fc_W, proj_W):
    x2 = x.reshape(L, H)
    p0, p1, w0, w1, bv, er, ue, nu, aux = pl.pallas_call(
        _router_body,
        out_shape=(
            jax.ShapeDtypeStruct((L, 1), jnp.int32),
            jax.ShapeDtypeStruct((L, 1), jnp.int32),
            jax.ShapeDtypeStruct((L, 16), jnp.float32),
            jax.ShapeDtypeStruct((L, 16), jnp.float32),
            jax.ShapeDtypeStruct((1, NB), jnp.int32),
            jax.ShapeDtypeStruct((1, NB), jnp.int32),
            jax.ShapeDtypeStruct((1, E), jnp.int32),
            jax.ShapeDtypeStruct((1, 1), jnp.int32),
            jax.ShapeDtypeStruct((1, 1), jnp.float32),
        ),
    )(x2, gate_W)
    p0f = p0.reshape(L)
    p1f = p1.reshape(L)

    _dispatch, _combine = _sc_kernels()
    xs = _dispatch(x2, p0f, p1f)

    ys = pl.pallas_call(
        _ffn_body,
        grid_spec=pltpu.PrefetchScalarGridSpec(
            num_scalar_prefetch=4,
            grid=(NB,),
            in_specs=[
                pl.BlockSpec((BT, H), lambda b, bv, er, ue, nu: (b, 0)),
                pl.BlockSpec(memory_space=pltpu.MemorySpace.HBM),
                pl.BlockSpec(memory_space=pltpu.MemorySpace.HBM),
            ],
            out_specs=pl.BlockSpec((BT, H), lambda b, bv, er, ue, nu: (b, 0)),
            scratch_shapes=[
                pltpu.VMEM((2, F, H), jnp.float32),
                pltpu.VMEM((2, H, F), jnp.float32),
                pltpu.SemaphoreType.DMA((2,)),
            ],
        ),
        out_shape=jax.ShapeDtypeStruct((PPAD, H), jnp.float32),
        compiler_params=pltpu.CompilerParams(
            dimension_semantics=("arbitrary",),
        ),
    )(bv, er, ue, nu, xs, fc_W, proj_W)

    out = _combine(ys, p0f, p1f, w0, w1)
    return out.reshape(B, L, H), aux.reshape(())
